# baseline (speedup 1.0000x reference)
import jax
import jax.numpy as jnp
from jax.experimental import pallas as pl


def _dinv_sqrt(idx, n):
    deg = jnp.zeros((n,), jnp.float32).at[idx].add(1.0)
    safe = jnp.where(deg > 0, deg, 1.0)
    return jnp.where(deg > 0, 1.0 / jnp.sqrt(safe), 0.0)


def _readout_body(pooled_ref, fc_W_ref, fc_b_ref, out_ref):
    logits = jnp.dot(pooled_ref[...], fc_W_ref[...],
                     preferred_element_type=jnp.float32) + fc_b_ref[...]
    m = jnp.max(logits, axis=1, keepdims=True)
    lse = m + jnp.log(jnp.sum(jnp.exp(logits - m), axis=1, keepdims=True))
    out_ref[...] = logits - lse


def kernel(x, edge_index, batch, cheb_W, cheb_b, gcn1_W, gcn1_b,
           gcn2_W, gcn2_b, gate_w, gate_b, fc_W, fc_b):
    n = x.shape[0]
    B = 8
    row, col = edge_index[0], edge_index[1]

    dinv = _dinv_sqrt(row, n)
    w = -(dinv[row] * dinv[col])

    def lhat(v):
        return jnp.zeros_like(v).at[col].add(w[:, None] * v[row])

    Tx0 = x
    Tx1 = lhat(x)
    out = Tx0 @ cheb_W[0] + Tx1 @ cheb_W[1]
    for k in range(2, 5):
        Tx2 = 2.0 * lhat(Tx1) - Tx0
        out = out + Tx2 @ cheb_W[k]
        Tx0, Tx1 = Tx1, Tx2
    h = jax.nn.relu(out + cheb_b)

    loop = jnp.arange(n, dtype=row.dtype)
    rows = jnp.concatenate([row, loop])
    cols = jnp.concatenate([col, loop])
    dgc = _dinv_sqrt(cols, n)
    norm = dgc[rows] * dgc[cols]

    def gcn(v, W, b):
        vw = v @ W
        agg = jnp.zeros((n, W.shape[1]), vw.dtype).at[cols].add(norm[:, None] * vw[rows])
        return agg + b

    h = jax.nn.relu(gcn(h, gcn1_W, gcn1_b))
    h = jax.nn.relu(gcn(h, gcn2_W, gcn2_b))

    g = (h @ gate_w + gate_b)[:, 0]
    gmax = jax.ops.segment_max(g, batch, num_segments=B)
    gmax = jnp.where(jnp.isfinite(gmax), gmax, 0.0)
    ge = jnp.exp(g - gmax[batch])
    gs = jax.ops.segment_sum(ge, batch, num_segments=B)
    att = ge / jnp.maximum(gs[batch], 1e-12)
    pooled = jax.ops.segment_sum(att[:, None] * h, batch, num_segments=B)

    C = fc_W.shape[1]
    return pl.pallas_call(
        _readout_body,
        out_shape=jax.ShapeDtypeStruct((B, C), jnp.float32),
    )(pooled, fc_W, fc_b.reshape(1, C))


# speedup vs baseline: 3.7828x; 3.7828x over previous
import functools

import jax
import jax.numpy as jnp
from jax.experimental import pallas as pl

NPAD = 10240
BM = 512
BK = 512


def _spmv_body(so_ref, si_ref, a_ref, v_ref, *rest, mode, out_ref=None):
    k = pl.program_id(1)
    nk = pl.num_programs(1)

    vb = v_ref[...] * si_ref[...]
    hi = vb.astype(jnp.bfloat16)
    lo = (vb - hi.astype(jnp.float32)).astype(jnp.bfloat16)
    a = a_ref[...]
    part = jnp.dot(a, hi, preferred_element_type=jnp.float32)
    part = part + jnp.dot(a, lo, preferred_element_type=jnp.float32)

    @pl.when(k == 0)
    def _():
        out_ref[...] = part

    @pl.when(k > 0)
    def _():
        out_ref[...] += part

    @pl.when(k == nk - 1)
    def _():
        base = out_ref[...] * so_ref[...]
        if mode == "scale":
            out_ref[...] = base
        elif mode == "cheb":
            aux_ref = rest[0]
            out_ref[...] = 2.0 * base - aux_ref[...]
        elif mode == "gcn":
            sl_ref, vown_ref, b_ref = rest
            out_ref[...] = jax.nn.relu(
                base + sl_ref[...] * vown_ref[...] + b_ref[...])


def _spmv(a, v, si, so, mode, aux=None, sl=None, b=None):
    w = v.shape[1]
    grid = (NPAD // BM, NPAD // BK)
    in_specs = [
        pl.BlockSpec((BM, 1), lambda i, k: (i, 0)),
        pl.BlockSpec((BK, 1), lambda i, k: (k, 0)),
        pl.BlockSpec((BM, BK), lambda i, k: (i, k)),
        pl.BlockSpec((BK, w), lambda i, k: (k, 0)),
    ]
    args = [so.reshape(NPAD, 1), si.reshape(NPAD, 1), a, v]
    if mode == "cheb":
        in_specs.append(pl.BlockSpec((BM, w), lambda i, k: (i, 0)))
        args.append(aux)
    elif mode == "gcn":
        in_specs.append(pl.BlockSpec((BM, 1), lambda i, k: (i, 0)))
        args.append(sl.reshape(NPAD, 1))
        in_specs.append(pl.BlockSpec((BM, w), lambda i, k: (i, 0)))
        args.append(v)
        in_specs.append(pl.BlockSpec((1, w), lambda i, k: (0, 0)))
        args.append(b.reshape(1, w))
    body = functools.partial(_spmv_body, mode=mode)

    def wrapped(*refs):
        body(*refs[:-1], out_ref=refs[-1])

    return pl.pallas_call(
        wrapped,
        grid=grid,
        in_specs=in_specs,
        out_specs=pl.BlockSpec((BM, w), lambda i, k: (i, 0)),
        out_shape=jax.ShapeDtypeStruct((NPAD, w), jnp.float32),
    )(*args)


def _mm_body(x_ref, w_ref, b_ref, out_ref, *, relu):
    r = jnp.dot(x_ref[...], w_ref[...],
                preferred_element_type=jnp.float32) + b_ref[...]
    out_ref[...] = jax.nn.relu(r) if relu else r


def _mm(x, w, b, relu):
    fin, fout = w.shape
    return pl.pallas_call(
        functools.partial(_mm_body, relu=relu),
        grid=(NPAD // BM,),
        in_specs=[
            pl.BlockSpec((BM, fin), lambda i: (i, 0)),
            pl.BlockSpec((fin, fout), lambda i: (0, 0)),
            pl.BlockSpec((1, fout), lambda i: (0, 0)),
        ],
        out_specs=pl.BlockSpec((BM, fout), lambda i: (i, 0)),
        out_shape=jax.ShapeDtypeStruct((NPAD, fout), jnp.float32),
    )(x, w, b.reshape(1, fout))


def _pool_body(h_ref, batch_ref, gate_w_ref, gate_b_ref, fc_w_ref, fc_b_ref,
               out_ref, *, nb):
    h = h_ref[...]
    g = jnp.dot(h, gate_w_ref[...],
                preferred_element_type=jnp.float32) + gate_b_ref[...]
    seg = jax.lax.broadcasted_iota(jnp.int32, (h.shape[0], nb), 1)
    m = batch_ref[...] == seg
    neg = jnp.float32(-jnp.inf)
    gmax = jnp.max(jnp.where(m, g, neg), axis=0, keepdims=True)
    gmax = jnp.where(jnp.isfinite(gmax), gmax, 0.0)
    ge = jnp.where(m, jnp.exp(g - gmax), 0.0)
    gs = jnp.sum(ge, axis=0, keepdims=True)
    att = ge / jnp.maximum(gs, 1e-12)
    pooled = jax.lax.dot_general(att, h, (((0,), (0,)), ((), ())),
                                 preferred_element_type=jnp.float32)
    logits = jnp.dot(pooled, fc_w_ref[...],
                     preferred_element_type=jnp.float32) + fc_b_ref[...]
    mx = jnp.max(logits, axis=1, keepdims=True)
    lse = mx + jnp.log(jnp.sum(jnp.exp(logits - mx), axis=1, keepdims=True))
    out_ref[...] = logits - lse


def _pool(h, batch_padded, gate_w, gate_b, fc_w, fc_b, nb):
    c = fc_w.shape[1]
    return pl.pallas_call(
        functools.partial(_pool_body, nb=nb),
        out_shape=jax.ShapeDtypeStruct((nb, c), jnp.float32),
    )(h, batch_padded.reshape(NPAD, 1), gate_w, gate_b.reshape(1, 1),
      fc_w, fc_b.reshape(1, c))


def kernel(x, edge_index, batch, cheb_W, cheb_b, gcn1_W, gcn1_b,
           gcn2_W, gcn2_b, gate_w, gate_b, fc_W, fc_b):
    n, f = x.shape
    nb = 8
    row, col = edge_index[0], edge_index[1]

    deg_r = jnp.zeros((NPAD,), jnp.float32).at[row].add(1.0)
    deg_c = jnp.zeros((NPAD,), jnp.float32).at[col].add(1.0)
    dinv = jnp.where(deg_r > 0, jax.lax.rsqrt(jnp.where(deg_r > 0, deg_r, 1.0)), 0.0)
    deg_gc = deg_c + 1.0
    dgc = jax.lax.rsqrt(deg_gc)
    dgc2 = dgc * dgc

    flat = col.astype(jnp.int32) * NPAD + row.astype(jnp.int32)
    a = (jnp.zeros((NPAD * NPAD,), jnp.bfloat16)
         .at[flat].add(jnp.bfloat16(1.0)).reshape(NPAD, NPAD))

    xp = jnp.zeros((NPAD, f), jnp.float32).at[:n].set(x)
    batch_p = jnp.full((NPAD,), nb, jnp.int32).at[:n].set(batch)

    tx0 = xp
    tx1 = _spmv(a, tx0, si=dinv, so=-dinv, mode="scale")
    tx2 = _spmv(a, tx1, si=dinv, so=-dinv, mode="cheb", aux=tx0)
    tx3 = _spmv(a, tx2, si=dinv, so=-dinv, mode="cheb", aux=tx1)
    tx4 = _spmv(a, tx3, si=dinv, so=-dinv, mode="cheb", aux=tx2)
    cat = jnp.concatenate([tx0, tx1, tx2, tx3, tx4], axis=1)
    wcat = cheb_W.reshape(5 * f, cheb_W.shape[2])
    h1 = _mm(cat, wcat, cheb_b, relu=True)

    vw1 = _mm(h1, gcn1_W, jnp.zeros((gcn1_W.shape[1],), jnp.float32), relu=False)
    h2 = _spmv(a, vw1, si=dgc, so=dgc, mode="gcn", sl=dgc2, b=gcn1_b)
    vw2 = _mm(h2, gcn2_W, jnp.zeros((gcn2_W.shape[1],), jnp.float32), relu=False)
    h3 = _spmv(a, vw2, si=dgc, so=dgc, mode="gcn", sl=dgc2, b=gcn2_b)

    return _pool(h3, batch_p, gate_w, gate_b, fc_W, fc_b, nb)


# speedup vs baseline: 5.2928x; 1.3992x over previous
import functools

import jax
import jax.numpy as jnp
from jax.experimental import pallas as pl

NPAD = 10240
BM = 512
BK = 512


def _spmv_body(so_ref, si_ref, a_ref, v_ref, *rest, mode, out_ref=None):
    k = pl.program_id(1)
    nk = pl.num_programs(1)

    vb = v_ref[...] * si_ref[...]
    hi = vb.astype(jnp.bfloat16)
    lo = (vb - hi.astype(jnp.float32)).astype(jnp.bfloat16)
    a = a_ref[...]
    part = jnp.dot(a, hi, preferred_element_type=jnp.float32)
    part = part + jnp.dot(a, lo, preferred_element_type=jnp.float32)

    @pl.when(k == 0)
    def _():
        out_ref[...] = part

    @pl.when(k > 0)
    def _():
        out_ref[...] += part

    @pl.when(k == nk - 1)
    def _():
        base = out_ref[...] * so_ref[...]
        if mode == "scale":
            out_ref[...] = base
        elif mode == "cheb":
            aux_ref = rest[0]
            out_ref[...] = 2.0 * base - aux_ref[...]
        elif mode == "gcn":
            sl_ref, vown_ref, b_ref = rest
            out_ref[...] = jax.nn.relu(
                base + sl_ref[...] * vown_ref[...] + b_ref[...])


def _spmv(a, v, si, so, mode, aux=None, sl=None, b=None):
    w = v.shape[1]
    grid = (NPAD // BM, NPAD // BK)
    in_specs = [
        pl.BlockSpec((BM, 1), lambda i, k: (i, 0)),
        pl.BlockSpec((BK, 1), lambda i, k: (k, 0)),
        pl.BlockSpec((BM, BK), lambda i, k: (i, k)),
        pl.BlockSpec((BK, w), lambda i, k: (k, 0)),
    ]
    args = [so.reshape(NPAD, 1), si.reshape(NPAD, 1), a, v]
    if mode == "cheb":
        in_specs.append(pl.BlockSpec((BM, w), lambda i, k: (i, 0)))
        args.append(aux)
    elif mode == "gcn":
        in_specs.append(pl.BlockSpec((BM, 1), lambda i, k: (i, 0)))
        args.append(sl.reshape(NPAD, 1))
        in_specs.append(pl.BlockSpec((BM, w), lambda i, k: (i, 0)))
        args.append(v)
        in_specs.append(pl.BlockSpec((1, w), lambda i, k: (0, 0)))
        args.append(b.reshape(1, w))
    body = functools.partial(_spmv_body, mode=mode)

    def wrapped(*refs):
        body(*refs[:-1], out_ref=refs[-1])

    return pl.pallas_call(
        wrapped,
        grid=grid,
        in_specs=in_specs,
        out_specs=pl.BlockSpec((BM, w), lambda i, k: (i, 0)),
        out_shape=jax.ShapeDtypeStruct((NPAD, w), jnp.float32),
    )(*args)


def _mm_body(x_ref, w_ref, b_ref, out_ref, *, relu):
    r = jnp.dot(x_ref[...], w_ref[...],
                preferred_element_type=jnp.float32) + b_ref[...]
    out_ref[...] = jax.nn.relu(r) if relu else r


def _mm(x, w, b, relu):
    fin, fout = w.shape
    return pl.pallas_call(
        functools.partial(_mm_body, relu=relu),
        grid=(NPAD // BM,),
        in_specs=[
            pl.BlockSpec((BM, fin), lambda i: (i, 0)),
            pl.BlockSpec((fin, fout), lambda i: (0, 0)),
            pl.BlockSpec((1, fout), lambda i: (0, 0)),
        ],
        out_specs=pl.BlockSpec((BM, fout), lambda i: (i, 0)),
        out_shape=jax.ShapeDtypeStruct((NPAD, fout), jnp.float32),
    )(x, w, b.reshape(1, fout))


def _pool_body(h_ref, batch_ref, gate_w_ref, gate_b_ref, fc_w_ref, fc_b_ref,
               out_ref, *, nb):
    h = h_ref[...]
    g = jnp.dot(h, gate_w_ref[...],
                preferred_element_type=jnp.float32) + gate_b_ref[...]
    seg = jax.lax.broadcasted_iota(jnp.int32, (h.shape[0], nb), 1)
    m = batch_ref[...] == seg
    neg = jnp.float32(-jnp.inf)
    gmax = jnp.max(jnp.where(m, g, neg), axis=0, keepdims=True)
    gmax = jnp.where(jnp.isfinite(gmax), gmax, 0.0)
    ge = jnp.where(m, jnp.exp(g - gmax), 0.0)
    gs = jnp.sum(ge, axis=0, keepdims=True)
    att = ge / jnp.maximum(gs, 1e-12)
    pooled = jax.lax.dot_general(att, h, (((0,), (0,)), ((), ())),
                                 preferred_element_type=jnp.float32)
    logits = jnp.dot(pooled, fc_w_ref[...],
                     preferred_element_type=jnp.float32) + fc_b_ref[...]
    mx = jnp.max(logits, axis=1, keepdims=True)
    lse = mx + jnp.log(jnp.sum(jnp.exp(logits - mx), axis=1, keepdims=True))
    out_ref[...] = logits - lse


def _pool(h, batch_padded, gate_w, gate_b, fc_w, fc_b, nb):
    c = fc_w.shape[1]
    return pl.pallas_call(
        functools.partial(_pool_body, nb=nb),
        out_shape=jax.ShapeDtypeStruct((nb, c), jnp.float32),
    )(h, batch_padded.reshape(NPAD, 1), gate_w, gate_b.reshape(1, 1),
      fc_w, fc_b.reshape(1, c))


def kernel(x, edge_index, batch, cheb_W, cheb_b, gcn1_W, gcn1_b,
           gcn2_W, gcn2_b, gate_w, gate_b, fc_W, fc_b):
    n, f = x.shape
    nb = 8
    row, col = edge_index[0], edge_index[1]

    deg_r = jnp.zeros((NPAD,), jnp.float32).at[row].add(1.0)
    deg_c = jnp.zeros((NPAD,), jnp.float32).at[col].add(1.0)
    dinv = jnp.where(deg_r > 0, jax.lax.rsqrt(jnp.where(deg_r > 0, deg_r, 1.0)), 0.0)
    deg_gc = deg_c + 1.0
    dgc = jax.lax.rsqrt(deg_gc)
    dgc2 = dgc * dgc

    flat = col.astype(jnp.int32) * NPAD + row.astype(jnp.int32)
    a = (jnp.zeros((NPAD * NPAD,), jnp.float32)
         .at[flat].add(1.0).reshape(NPAD, NPAD).astype(jnp.bfloat16))

    xp = jnp.zeros((NPAD, f), jnp.float32).at[:n].set(x)
    batch_p = jnp.full((NPAD,), nb, jnp.int32).at[:n].set(batch)

    tx0 = xp
    tx1 = _spmv(a, tx0, si=dinv, so=-dinv, mode="scale")
    tx2 = _spmv(a, tx1, si=dinv, so=-dinv, mode="cheb", aux=tx0)
    tx3 = _spmv(a, tx2, si=dinv, so=-dinv, mode="cheb", aux=tx1)
    tx4 = _spmv(a, tx3, si=dinv, so=-dinv, mode="cheb", aux=tx2)
    cat = jnp.concatenate([tx0, tx1, tx2, tx3, tx4], axis=1)
    wcat = cheb_W.reshape(5 * f, cheb_W.shape[2])
    h1 = _mm(cat, wcat, cheb_b, relu=True)

    vw1 = _mm(h1, gcn1_W, jnp.zeros((gcn1_W.shape[1],), jnp.float32), relu=False)
    h2 = _spmv(a, vw1, si=dgc, so=dgc, mode="gcn", sl=dgc2, b=gcn1_b)
    vw2 = _mm(h2, gcn2_W, jnp.zeros((gcn2_W.shape[1],), jnp.float32), relu=False)
    h3 = _spmv(a, vw2, si=dgc, so=dgc, mode="gcn", sl=dgc2, b=gcn2_b)

    return _pool(h3, batch_p, gate_w, gate_b, fc_W, fc_b, nb)


# speedup vs baseline: 7.5039x; 1.4178x over previous
import functools

import jax
import jax.numpy as jnp
from jax.experimental import pallas as pl

NPAD = 10240
BM = 1024
BK = 1024


def _split(s):
    hi = s.astype(jnp.bfloat16)
    lo = (s - hi.astype(jnp.float32)).astype(jnp.bfloat16)
    return hi, lo


def _prep_body(v_ref, si_ref, hi_ref, lo_ref):
    hi, lo = _split(v_ref[...] * si_ref[...])
    hi_ref[...] = hi
    lo_ref[...] = lo


def _prep(v, si):
    w = v.shape[1]
    return pl.pallas_call(
        _prep_body,
        grid=(NPAD // BM,),
        in_specs=[
            pl.BlockSpec((BM, w), lambda i: (i, 0)),
            pl.BlockSpec((BM, 1), lambda i: (i, 0)),
        ],
        out_specs=[pl.BlockSpec((BM, w), lambda i: (i, 0))] * 2,
        out_shape=[jax.ShapeDtypeStruct((NPAD, w), jnp.bfloat16)] * 2,
    )(v, si.reshape(NPAD, 1))


def _spmv_body(*refs, mode, emit):
    k = pl.program_id(1)
    nk = pl.num_programs(1)
    if mode == "gcn":
        (so_ref, a_ref, hi_ref, lo_ref, sl_ref, vown_ref, b_ref), rest = \
            refs[:7], refs[7:]
    elif mode == "cheb":
        (so_ref, a_ref, hi_ref, lo_ref, aux_ref), rest = refs[:5], refs[5:]
    else:
        (so_ref, a_ref, hi_ref, lo_ref), rest = refs[:4], refs[4:]
    if emit:
        sin_ref = rest[0]
        out_ref, ohi_ref, olo_ref = rest[1:]
    else:
        (out_ref,) = rest

    a = a_ref[...]
    part = jnp.dot(a, hi_ref[...], preferred_element_type=jnp.float32)
    part = part + jnp.dot(a, lo_ref[...], preferred_element_type=jnp.float32)

    @pl.when(k == 0)
    def _():
        out_ref[...] = part

    @pl.when(k > 0)
    def _():
        out_ref[...] += part

    @pl.when(k == nk - 1)
    def _():
        base = out_ref[...] * so_ref[...]
        if mode == "scale":
            res = base
        elif mode == "cheb":
            res = 2.0 * base - aux_ref[...]
        else:
            res = jax.nn.relu(base + sl_ref[...] * vown_ref[...] + b_ref[...])
        out_ref[...] = res
        if emit:
            hi, lo = _split(res * sin_ref[...])
            ohi_ref[...] = hi
            olo_ref[...] = lo


def _spmv(a, hi, lo, so, mode, aux=None, sl=None, vown=None, b=None,
          si_next=None):
    w = hi.shape[1]
    grid = (NPAD // BM, NPAD // BK)
    emit = si_next is not None
    in_specs = [
        pl.BlockSpec((BM, 1), lambda i, k: (i, 0)),
        pl.BlockSpec((BM, BK), lambda i, k: (i, k)),
        pl.BlockSpec((BK, w), lambda i, k: (k, 0)),
        pl.BlockSpec((BK, w), lambda i, k: (k, 0)),
    ]
    args = [so.reshape(NPAD, 1), a, hi, lo]
    if mode == "cheb":
        in_specs.append(pl.BlockSpec((BM, w), lambda i, k: (i, 0)))
        args.append(aux)
    elif mode == "gcn":
        in_specs += [
            pl.BlockSpec((BM, 1), lambda i, k: (i, 0)),
            pl.BlockSpec((BM, w), lambda i, k: (i, 0)),
            pl.BlockSpec((1, w), lambda i, k: (0, 0)),
        ]
        args += [sl.reshape(NPAD, 1), vown, b.reshape(1, w)]
    if emit:
        in_specs.append(pl.BlockSpec((BM, 1), lambda i, k: (i, 0)))
        args.append(si_next.reshape(NPAD, 1))
        out_specs = [pl.BlockSpec((BM, w), lambda i, k: (i, 0))] * 3
        out_shape = [jax.ShapeDtypeStruct((NPAD, w), jnp.float32),
                     jax.ShapeDtypeStruct((NPAD, w), jnp.bfloat16),
                     jax.ShapeDtypeStruct((NPAD, w), jnp.bfloat16)]
    else:
        out_specs = pl.BlockSpec((BM, w), lambda i, k: (i, 0))
        out_shape = jax.ShapeDtypeStruct((NPAD, w), jnp.float32)

    return pl.pallas_call(
        functools.partial(_spmv_body, mode=mode, emit=emit),
        grid=grid,
        in_specs=in_specs,
        out_specs=out_specs,
        out_shape=out_shape,
    )(*args)


def _mm_body(*refs, relu, emit):
    if emit:
        x_ref, w_ref, b_ref, si_ref, out_ref, hi_ref, lo_ref = refs
    else:
        x_ref, w_ref, b_ref, out_ref = refs
    r = jnp.dot(x_ref[...], w_ref[...],
                preferred_element_type=jnp.float32) + b_ref[...]
    if relu:
        r = jax.nn.relu(r)
    out_ref[...] = r
    if emit:
        hi, lo = _split(r * si_ref[...])
        hi_ref[...] = hi
        lo_ref[...] = lo


def _mm(x, w, b, relu, si_next=None):
    fin, fout = w.shape
    emit = si_next is not None
    in_specs = [
        pl.BlockSpec((BM, fin), lambda i: (i, 0)),
        pl.BlockSpec((fin, fout), lambda i: (0, 0)),
        pl.BlockSpec((1, fout), lambda i: (0, 0)),
    ]
    args = [x, w, b.reshape(1, fout)]
    if emit:
        in_specs.append(pl.BlockSpec((BM, 1), lambda i: (i, 0)))
        args.append(si_next.reshape(NPAD, 1))
        out_specs = [pl.BlockSpec((BM, fout), lambda i: (i, 0))] * 3
        out_shape = [jax.ShapeDtypeStruct((NPAD, fout), jnp.float32),
                     jax.ShapeDtypeStruct((NPAD, fout), jnp.bfloat16),
                     jax.ShapeDtypeStruct((NPAD, fout), jnp.bfloat16)]
    else:
        out_specs = pl.BlockSpec((BM, fout), lambda i: (i, 0))
        out_shape = jax.ShapeDtypeStruct((NPAD, fout), jnp.float32)
    return pl.pallas_call(
        functools.partial(_mm_body, relu=relu, emit=emit),
        grid=(NPAD // BM,),
        in_specs=in_specs,
        out_specs=out_specs,
        out_shape=out_shape,
    )(*args)


def _pool_body(h_ref, batch_ref, gate_w_ref, gate_b_ref, fc_w_ref, fc_b_ref,
               out_ref, *, nb):
    h = h_ref[...]
    g = jnp.dot(h, gate_w_ref[...],
                preferred_element_type=jnp.float32) + gate_b_ref[...]
    seg = jax.lax.broadcasted_iota(jnp.int32, (h.shape[0], nb), 1)
    m = batch_ref[...] == seg
    neg = jnp.float32(-jnp.inf)
    gmax = jnp.max(jnp.where(m, g, neg), axis=0, keepdims=True)
    gmax = jnp.where(jnp.isfinite(gmax), gmax, 0.0)
    ge = jnp.where(m, jnp.exp(g - gmax), 0.0)
    gs = jnp.sum(ge, axis=0, keepdims=True)
    att = ge / jnp.maximum(gs, 1e-12)
    pooled = jax.lax.dot_general(att, h, (((0,), (0,)), ((), ())),
                                 preferred_element_type=jnp.float32)
    logits = jnp.dot(pooled, fc_w_ref[...],
                     preferred_element_type=jnp.float32) + fc_b_ref[...]
    mx = jnp.max(logits, axis=1, keepdims=True)
    lse = mx + jnp.log(jnp.sum(jnp.exp(logits - mx), axis=1, keepdims=True))
    out_ref[...] = logits - lse


def _pool(h, batch_padded, gate_w, gate_b, fc_w, fc_b, nb):
    c = fc_w.shape[1]
    return pl.pallas_call(
        functools.partial(_pool_body, nb=nb),
        out_shape=jax.ShapeDtypeStruct((nb, c), jnp.float32),
    )(h, batch_padded.reshape(NPAD, 1), gate_w, gate_b.reshape(1, 1),
      fc_w, fc_b.reshape(1, c))


def kernel(x, edge_index, batch, cheb_W, cheb_b, gcn1_W, gcn1_b,
           gcn2_W, gcn2_b, gate_w, gate_b, fc_W, fc_b):
    n, f = x.shape
    nb = 8
    row, col = edge_index[0], edge_index[1]

    deg_r = jnp.zeros((NPAD,), jnp.float32).at[row].add(1.0)
    deg_c = jnp.zeros((NPAD,), jnp.float32).at[col].add(1.0)
    dinv = jnp.where(deg_r > 0, jax.lax.rsqrt(jnp.where(deg_r > 0, deg_r, 1.0)), 0.0)
    dgc = jax.lax.rsqrt(deg_c + 1.0)
    dgc2 = dgc * dgc

    flat = col.astype(jnp.int32) * NPAD + row.astype(jnp.int32)
    a = (jnp.zeros((NPAD * NPAD,), jnp.float32)
         .at[flat].add(1.0).reshape(NPAD, NPAD).astype(jnp.bfloat16))

    xp = jnp.zeros((NPAD, f), jnp.float32).at[:n].set(x)
    batch_p = jnp.full((NPAD,), nb, jnp.int32).at[:n].set(batch)

    tx0 = xp
    h0, l0 = _prep(tx0, dinv)
    tx1, h1_, l1_ = _spmv(a, h0, l0, so=-dinv, mode="scale", si_next=dinv)
    tx2, h2_, l2_ = _spmv(a, h1_, l1_, so=-dinv, mode="cheb", aux=tx0,
                          si_next=dinv)
    tx3, h3_, l3_ = _spmv(a, h2_, l2_, so=-dinv, mode="cheb", aux=tx1,
                          si_next=dinv)
    tx4 = _spmv(a, h3_, l3_, so=-dinv, mode="cheb", aux=tx2)
    cat = jnp.concatenate([tx0, tx1, tx2, tx3, tx4], axis=1)
    wcat = cheb_W.reshape(5 * f, cheb_W.shape[2])
    h1 = _mm(cat, wcat, cheb_b, relu=True)

    z1 = jnp.zeros((gcn1_W.shape[1],), jnp.float32)
    z2 = jnp.zeros((gcn2_W.shape[1],), jnp.float32)
    vw1, vh1, vl1 = _mm(h1, gcn1_W, z1, relu=False, si_next=dgc)
    h2 = _spmv(a, vh1, vl1, so=dgc, mode="gcn", sl=dgc2, vown=vw1, b=gcn1_b)
    vw2, vh2, vl2 = _mm(h2, gcn2_W, z2, relu=False, si_next=dgc)
    h3 = _spmv(a, vh2, vl2, so=dgc, mode="gcn", sl=dgc2, vown=vw2, b=gcn2_b)

    return _pool(h3, batch_p, gate_w, gate_b, fc_W, fc_b, nb)


# speedup vs baseline: 7.7682x; 1.0352x over previous
import functools

import jax
import jax.numpy as jnp
from jax.experimental import pallas as pl

NPAD = 10240
BM = 1024
BK = 1024


def _split(s):
    hi = s.astype(jnp.bfloat16)
    lo = (s - hi.astype(jnp.float32)).astype(jnp.bfloat16)
    return hi, lo


def _prep_body(v_ref, si_ref, hl_ref):
    hi, lo = _split(v_ref[...] * si_ref[...])
    hl_ref[...] = jnp.concatenate([hi, lo], axis=1)


def _prep(v, si):
    w = v.shape[1]
    return pl.pallas_call(
        _prep_body,
        grid=(NPAD // BM,),
        in_specs=[
            pl.BlockSpec((BM, w), lambda i: (i, 0)),
            pl.BlockSpec((BM, 1), lambda i: (i, 0)),
        ],
        out_specs=pl.BlockSpec((BM, 2 * w), lambda i: (i, 0)),
        out_shape=jax.ShapeDtypeStruct((NPAD, 2 * w), jnp.bfloat16),
    )(v, si.reshape(NPAD, 1))


def _spmv_body(*refs, mode, emit):
    k = pl.program_id(1)
    nk = pl.num_programs(1)
    if mode == "gcn":
        (so_ref, a_ref, hl_ref, sl_ref, vown_ref, b_ref), rest = \
            refs[:6], refs[6:]
    elif mode == "cheb":
        (so_ref, a_ref, hl_ref, aux_ref), rest = refs[:4], refs[4:]
    else:
        (so_ref, a_ref, hl_ref), rest = refs[:3], refs[3:]
    if emit:
        sin_ref = rest[0]
        out_ref, ohl_ref = rest[1:]
    else:
        (out_ref,) = rest

    w = hl_ref.shape[1] // 2
    both = jnp.dot(a_ref[...], hl_ref[...], preferred_element_type=jnp.float32)
    part = both[:, :w] + both[:, w:]

    @pl.when(k == 0)
    def _():
        out_ref[...] = part

    @pl.when(k > 0)
    def _():
        out_ref[...] += part

    @pl.when(k == nk - 1)
    def _():
        base = out_ref[...] * so_ref[...]
        if mode == "scale":
            res = base
        elif mode == "cheb":
            res = 2.0 * base - aux_ref[...]
        else:
            res = jax.nn.relu(base + sl_ref[...] * vown_ref[...] + b_ref[...])
        out_ref[...] = res
        if emit:
            hi, lo = _split(res * sin_ref[...])
            ohl_ref[...] = jnp.concatenate([hi, lo], axis=1)


def _spmv(a, hl, so, mode, aux=None, sl=None, vown=None, b=None,
          si_next=None):
    w = hl.shape[1] // 2
    grid = (NPAD // BM, NPAD // BK)
    emit = si_next is not None
    in_specs = [
        pl.BlockSpec((BM, 1), lambda i, k: (i, 0)),
        pl.BlockSpec((BM, BK), lambda i, k: (i, k)),
        pl.BlockSpec((BK, 2 * w), lambda i, k: (k, 0)),
    ]
    args = [so.reshape(NPAD, 1), a, hl]
    if mode == "cheb":
        in_specs.append(pl.BlockSpec((BM, w), lambda i, k: (i, 0)))
        args.append(aux)
    elif mode == "gcn":
        in_specs += [
            pl.BlockSpec((BM, 1), lambda i, k: (i, 0)),
            pl.BlockSpec((BM, w), lambda i, k: (i, 0)),
            pl.BlockSpec((1, w), lambda i, k: (0, 0)),
        ]
        args += [sl.reshape(NPAD, 1), vown, b.reshape(1, w)]
    if emit:
        in_specs.append(pl.BlockSpec((BM, 1), lambda i, k: (i, 0)))
        args.append(si_next.reshape(NPAD, 1))
        out_specs = [pl.BlockSpec((BM, w), lambda i, k: (i, 0)),
                     pl.BlockSpec((BM, 2 * w), lambda i, k: (i, 0))]
        out_shape = [jax.ShapeDtypeStruct((NPAD, w), jnp.float32),
                     jax.ShapeDtypeStruct((NPAD, 2 * w), jnp.bfloat16)]
    else:
        out_specs = pl.BlockSpec((BM, w), lambda i, k: (i, 0))
        out_shape = jax.ShapeDtypeStruct((NPAD, w), jnp.float32)

    return pl.pallas_call(
        functools.partial(_spmv_body, mode=mode, emit=emit),
        grid=grid,
        in_specs=in_specs,
        out_specs=out_specs,
        out_shape=out_shape,
    )(*args)


def _mm_body(*refs, relu, emit):
    if emit:
        x_ref, w_ref, b_ref, si_ref, out_ref, ohl_ref = refs
    else:
        x_ref, w_ref, b_ref, out_ref = refs
    r = jnp.dot(x_ref[...], w_ref[...],
                preferred_element_type=jnp.float32) + b_ref[...]
    if relu:
        r = jax.nn.relu(r)
    out_ref[...] = r
    if emit:
        hi, lo = _split(r * si_ref[...])
        ohl_ref[...] = jnp.concatenate([hi, lo], axis=1)


def _mm(x, w, b, relu, si_next=None):
    fin, fout = w.shape
    emit = si_next is not None
    in_specs = [
        pl.BlockSpec((BM, fin), lambda i: (i, 0)),
        pl.BlockSpec((fin, fout), lambda i: (0, 0)),
        pl.BlockSpec((1, fout), lambda i: (0, 0)),
    ]
    args = [x, w, b.reshape(1, fout)]
    if emit:
        in_specs.append(pl.BlockSpec((BM, 1), lambda i: (i, 0)))
        args.append(si_next.reshape(NPAD, 1))
        out_specs = [pl.BlockSpec((BM, fout), lambda i: (i, 0)),
                     pl.BlockSpec((BM, 2 * fout), lambda i: (i, 0))]
        out_shape = [jax.ShapeDtypeStruct((NPAD, fout), jnp.float32),
                     jax.ShapeDtypeStruct((NPAD, 2 * fout), jnp.bfloat16)]
    else:
        out_specs = pl.BlockSpec((BM, fout), lambda i: (i, 0))
        out_shape = jax.ShapeDtypeStruct((NPAD, fout), jnp.float32)
    return pl.pallas_call(
        functools.partial(_mm_body, relu=relu, emit=emit),
        grid=(NPAD // BM,),
        in_specs=in_specs,
        out_specs=out_specs,
        out_shape=out_shape,
    )(*args)


def _pool_body(h_ref, batch_ref, gate_w_ref, gate_b_ref, fc_w_ref, fc_b_ref,
               out_ref, *, nb):
    h = h_ref[...]
    g = jnp.dot(h, gate_w_ref[...],
                preferred_element_type=jnp.float32) + gate_b_ref[...]
    seg = jax.lax.broadcasted_iota(jnp.int32, (h.shape[0], nb), 1)
    m = batch_ref[...] == seg
    neg = jnp.float32(-jnp.inf)
    gmax = jnp.max(jnp.where(m, g, neg), axis=0, keepdims=True)
    gmax = jnp.where(jnp.isfinite(gmax), gmax, 0.0)
    ge = jnp.where(m, jnp.exp(g - gmax), 0.0)
    gs = jnp.sum(ge, axis=0, keepdims=True)
    att = ge / jnp.maximum(gs, 1e-12)
    pooled = jax.lax.dot_general(att, h, (((0,), (0,)), ((), ())),
                                 preferred_element_type=jnp.float32)
    logits = jnp.dot(pooled, fc_w_ref[...],
                     preferred_element_type=jnp.float32) + fc_b_ref[...]
    mx = jnp.max(logits, axis=1, keepdims=True)
    lse = mx + jnp.log(jnp.sum(jnp.exp(logits - mx), axis=1, keepdims=True))
    out_ref[...] = logits - lse


def _pool(h, batch_padded, gate_w, gate_b, fc_w, fc_b, nb):
    c = fc_w.shape[1]
    return pl.pallas_call(
        functools.partial(_pool_body, nb=nb),
        out_shape=jax.ShapeDtypeStruct((nb, c), jnp.float32),
    )(h, batch_padded.reshape(NPAD, 1), gate_w, gate_b.reshape(1, 1),
      fc_w, fc_b.reshape(1, c))


def kernel(x, edge_index, batch, cheb_W, cheb_b, gcn1_W, gcn1_b,
           gcn2_W, gcn2_b, gate_w, gate_b, fc_W, fc_b):
    n, f = x.shape
    nb = 8
    row, col = edge_index[0], edge_index[1]

    deg_r = jnp.zeros((NPAD,), jnp.float32).at[row].add(1.0)
    deg_c = jnp.zeros((NPAD,), jnp.float32).at[col].add(1.0)
    dinv = jnp.where(deg_r > 0, jax.lax.rsqrt(jnp.where(deg_r > 0, deg_r, 1.0)), 0.0)
    dgc = jax.lax.rsqrt(deg_c + 1.0)
    dgc2 = dgc * dgc

    flat = col.astype(jnp.int32) * NPAD + row.astype(jnp.int32)
    a = (jnp.zeros((NPAD * NPAD,), jnp.float32)
         .at[flat].add(1.0).reshape(NPAD, NPAD).astype(jnp.bfloat16))

    xp = jnp.zeros((NPAD, f), jnp.float32).at[:n].set(x)
    batch_p = jnp.full((NPAD,), nb, jnp.int32).at[:n].set(batch)

    tx0 = xp
    hl0 = _prep(tx0, dinv)
    tx1, hl1 = _spmv(a, hl0, so=-dinv, mode="scale", si_next=dinv)
    tx2, hl2 = _spmv(a, hl1, so=-dinv, mode="cheb", aux=tx0, si_next=dinv)
    tx3, hl3 = _spmv(a, hl2, so=-dinv, mode="cheb", aux=tx1, si_next=dinv)
    tx4 = _spmv(a, hl3, so=-dinv, mode="cheb", aux=tx2)
    cat = jnp.concatenate([tx0, tx1, tx2, tx3, tx4], axis=1)
    wcat = cheb_W.reshape(5 * f, cheb_W.shape[2])
    h1 = _mm(cat, wcat, cheb_b, relu=True)

    z1 = jnp.zeros((gcn1_W.shape[1],), jnp.float32)
    z2 = jnp.zeros((gcn2_W.shape[1],), jnp.float32)
    vw1, vhl1 = _mm(h1, gcn1_W, z1, relu=False, si_next=dgc)
    h2 = _spmv(a, vhl1, so=dgc, mode="gcn", sl=dgc2, vown=vw1, b=gcn1_b)
    vw2, vhl2 = _mm(h2, gcn2_W, z2, relu=False, si_next=dgc)
    h3 = _spmv(a, vhl2, so=dgc, mode="gcn", sl=dgc2, vown=vw2, b=gcn2_b)

    return _pool(h3, batch_p, gate_w, gate_b, fc_W, fc_b, nb)


# speedup vs baseline: 8.2817x; 1.0661x over previous
import functools

import jax
import jax.numpy as jnp
from jax.experimental import pallas as pl

NPAD = 10240
BM = 1024
BK = 2048


def _split(s):
    hi = s.astype(jnp.bfloat16)
    lo = (s - hi.astype(jnp.float32)).astype(jnp.bfloat16)
    return hi, lo


def _prep_body(v_ref, si_ref, hl_ref):
    hi, lo = _split(v_ref[...] * si_ref[...])
    hl_ref[...] = jnp.concatenate([hi, lo], axis=1)


def _prep(v, si):
    w = v.shape[1]
    return pl.pallas_call(
        _prep_body,
        grid=(NPAD // BM,),
        in_specs=[
            pl.BlockSpec((BM, w), lambda i: (i, 0)),
            pl.BlockSpec((BM, 1), lambda i: (i, 0)),
        ],
        out_specs=pl.BlockSpec((BM, 2 * w), lambda i: (i, 0)),
        out_shape=jax.ShapeDtypeStruct((NPAD, 2 * w), jnp.bfloat16),
    )(v, si.reshape(NPAD, 1))


def _spmv_body(*refs, mode, emit):
    k = pl.program_id(1)
    nk = pl.num_programs(1)
    if mode == "gcn":
        (so_ref, a_ref, hl_ref, sl_ref, vown_ref, b_ref), rest = \
            refs[:6], refs[6:]
    elif mode == "cheb":
        (so_ref, a_ref, hl_ref, aux_ref), rest = refs[:4], refs[4:]
    else:
        (so_ref, a_ref, hl_ref), rest = refs[:3], refs[3:]
    if emit:
        sin_ref = rest[0]
        out_ref, ohl_ref = rest[1:]
    else:
        (out_ref,) = rest

    w = hl_ref.shape[1] // 2
    both = jnp.dot(a_ref[...], hl_ref[...], preferred_element_type=jnp.float32)
    part = both[:, :w] + both[:, w:]

    @pl.when(k == 0)
    def _():
        out_ref[...] = part

    @pl.when(k > 0)
    def _():
        out_ref[...] += part

    @pl.when(k == nk - 1)
    def _():
        base = out_ref[...] * so_ref[...]
        if mode == "scale":
            res = base
        elif mode == "cheb":
            res = 2.0 * base - aux_ref[...]
        else:
            res = jax.nn.relu(base + sl_ref[...] * vown_ref[...] + b_ref[...])
        out_ref[...] = res
        if emit:
            hi, lo = _split(res * sin_ref[...])
            ohl_ref[...] = jnp.concatenate([hi, lo], axis=1)


def _spmv(a, hl, so, mode, aux=None, sl=None, vown=None, b=None,
          si_next=None):
    w = hl.shape[1] // 2
    grid = (NPAD // BM, NPAD // BK)
    emit = si_next is not None
    in_specs = [
        pl.BlockSpec((BM, 1), lambda i, k: (i, 0)),
        pl.BlockSpec((BM, BK), lambda i, k: (i, k)),
        pl.BlockSpec((BK, 2 * w), lambda i, k: (k, 0)),
    ]
    args = [so.reshape(NPAD, 1), a, hl]
    if mode == "cheb":
        in_specs.append(pl.BlockSpec((BM, w), lambda i, k: (i, 0)))
        args.append(aux)
    elif mode == "gcn":
        in_specs += [
            pl.BlockSpec((BM, 1), lambda i, k: (i, 0)),
            pl.BlockSpec((BM, w), lambda i, k: (i, 0)),
            pl.BlockSpec((1, w), lambda i, k: (0, 0)),
        ]
        args += [sl.reshape(NPAD, 1), vown, b.reshape(1, w)]
    if emit:
        in_specs.append(pl.BlockSpec((BM, 1), lambda i, k: (i, 0)))
        args.append(si_next.reshape(NPAD, 1))
        out_specs = [pl.BlockSpec((BM, w), lambda i, k: (i, 0)),
                     pl.BlockSpec((BM, 2 * w), lambda i, k: (i, 0))]
        out_shape = [jax.ShapeDtypeStruct((NPAD, w), jnp.float32),
                     jax.ShapeDtypeStruct((NPAD, 2 * w), jnp.bfloat16)]
    else:
        out_specs = pl.BlockSpec((BM, w), lambda i, k: (i, 0))
        out_shape = jax.ShapeDtypeStruct((NPAD, w), jnp.float32)

    return pl.pallas_call(
        functools.partial(_spmv_body, mode=mode, emit=emit),
        grid=grid,
        in_specs=in_specs,
        out_specs=out_specs,
        out_shape=out_shape,
    )(*args)


def _mm_body(*refs, relu, emit):
    if emit:
        x_ref, w_ref, b_ref, si_ref, out_ref, ohl_ref = refs
    else:
        x_ref, w_ref, b_ref, out_ref = refs
    r = jnp.dot(x_ref[...], w_ref[...],
                preferred_element_type=jnp.float32) + b_ref[...]
    if relu:
        r = jax.nn.relu(r)
    out_ref[...] = r
    if emit:
        hi, lo = _split(r * si_ref[...])
        ohl_ref[...] = jnp.concatenate([hi, lo], axis=1)


def _mm(x, w, b, relu, si_next=None):
    fin, fout = w.shape
    emit = si_next is not None
    in_specs = [
        pl.BlockSpec((BM, fin), lambda i: (i, 0)),
        pl.BlockSpec((fin, fout), lambda i: (0, 0)),
        pl.BlockSpec((1, fout), lambda i: (0, 0)),
    ]
    args = [x, w, b.reshape(1, fout)]
    if emit:
        in_specs.append(pl.BlockSpec((BM, 1), lambda i: (i, 0)))
        args.append(si_next.reshape(NPAD, 1))
        out_specs = [pl.BlockSpec((BM, fout), lambda i: (i, 0)),
                     pl.BlockSpec((BM, 2 * fout), lambda i: (i, 0))]
        out_shape = [jax.ShapeDtypeStruct((NPAD, fout), jnp.float32),
                     jax.ShapeDtypeStruct((NPAD, 2 * fout), jnp.bfloat16)]
    else:
        out_specs = pl.BlockSpec((BM, fout), lambda i: (i, 0))
        out_shape = jax.ShapeDtypeStruct((NPAD, fout), jnp.float32)
    return pl.pallas_call(
        functools.partial(_mm_body, relu=relu, emit=emit),
        grid=(NPAD // BM,),
        in_specs=in_specs,
        out_specs=out_specs,
        out_shape=out_shape,
    )(*args)


def _pool_body(h_ref, batch_ref, gate_w_ref, gate_b_ref, fc_w_ref, fc_b_ref,
               out_ref, *, nb):
    h = h_ref[...]
    g = jnp.dot(h, gate_w_ref[...],
                preferred_element_type=jnp.float32) + gate_b_ref[...]
    seg = jax.lax.broadcasted_iota(jnp.int32, (h.shape[0], nb), 1)
    m = batch_ref[...] == seg
    neg = jnp.float32(-jnp.inf)
    gmax = jnp.max(jnp.where(m, g, neg), axis=0, keepdims=True)
    gmax = jnp.where(jnp.isfinite(gmax), gmax, 0.0)
    ge = jnp.where(m, jnp.exp(g - gmax), 0.0)
    gs = jnp.sum(ge, axis=0, keepdims=True)
    att = ge / jnp.maximum(gs, 1e-12)
    pooled = jax.lax.dot_general(att, h, (((0,), (0,)), ((), ())),
                                 preferred_element_type=jnp.float32)
    logits = jnp.dot(pooled, fc_w_ref[...],
                     preferred_element_type=jnp.float32) + fc_b_ref[...]
    mx = jnp.max(logits, axis=1, keepdims=True)
    lse = mx + jnp.log(jnp.sum(jnp.exp(logits - mx), axis=1, keepdims=True))
    out_ref[...] = logits - lse


def _pool(h, batch_padded, gate_w, gate_b, fc_w, fc_b, nb):
    c = fc_w.shape[1]
    return pl.pallas_call(
        functools.partial(_pool_body, nb=nb),
        out_shape=jax.ShapeDtypeStruct((nb, c), jnp.float32),
    )(h, batch_padded.reshape(NPAD, 1), gate_w, gate_b.reshape(1, 1),
      fc_w, fc_b.reshape(1, c))


def kernel(x, edge_index, batch, cheb_W, cheb_b, gcn1_W, gcn1_b,
           gcn2_W, gcn2_b, gate_w, gate_b, fc_W, fc_b):
    n, f = x.shape
    nb = 8
    row, col = edge_index[0], edge_index[1]

    deg_r = jnp.zeros((NPAD,), jnp.float32).at[row].add(1.0)
    deg_c = jnp.zeros((NPAD,), jnp.float32).at[col].add(1.0)
    dinv = jnp.where(deg_r > 0, jax.lax.rsqrt(jnp.where(deg_r > 0, deg_r, 1.0)), 0.0)
    dgc = jax.lax.rsqrt(deg_c + 1.0)
    dgc2 = dgc * dgc

    flat = col.astype(jnp.int32) * NPAD + row.astype(jnp.int32)
    a = (jnp.zeros((NPAD * NPAD,), jnp.float32)
         .at[flat].add(1.0).reshape(NPAD, NPAD).astype(jnp.bfloat16))

    xp = jnp.zeros((NPAD, f), jnp.float32).at[:n].set(x)
    batch_p = jnp.full((NPAD,), nb, jnp.int32).at[:n].set(batch)

    tx0 = xp
    hl0 = _prep(tx0, dinv)
    tx1, hl1 = _spmv(a, hl0, so=-dinv, mode="scale", si_next=dinv)
    tx2, hl2 = _spmv(a, hl1, so=-dinv, mode="cheb", aux=tx0, si_next=dinv)
    tx3, hl3 = _spmv(a, hl2, so=-dinv, mode="cheb", aux=tx1, si_next=dinv)
    tx4 = _spmv(a, hl3, so=-dinv, mode="cheb", aux=tx2)
    cat = jnp.concatenate([tx0, tx1, tx2, tx3, tx4], axis=1)
    wcat = cheb_W.reshape(5 * f, cheb_W.shape[2])
    h1 = _mm(cat, wcat, cheb_b, relu=True)

    z1 = jnp.zeros((gcn1_W.shape[1],), jnp.float32)
    z2 = jnp.zeros((gcn2_W.shape[1],), jnp.float32)
    vw1, vhl1 = _mm(h1, gcn1_W, z1, relu=False, si_next=dgc)
    h2 = _spmv(a, vhl1, so=dgc, mode="gcn", sl=dgc2, vown=vw1, b=gcn1_b)
    vw2, vhl2 = _mm(h2, gcn2_W, z2, relu=False, si_next=dgc)
    h3 = _spmv(a, vhl2, so=dgc, mode="gcn", sl=dgc2, vown=vw2, b=gcn2_b)

    return _pool(h3, batch_p, gate_w, gate_b, fc_W, fc_b, nb)
